# Initial kernel scaffold; baseline (speedup 1.0000x reference)
import math

import jax
import jax.numpy as jnp
from jax import lax
from jax.experimental import pallas as pl
from jax.experimental.pallas import tpu as pltpu
from jax.experimental.pallas import tpu_sc as plsc

N = 10000
E = 320000
NC = 2
NS = 16
NW = NC * NS
CH = 128
ACC_ROWS = 10016
ZROWS = ACC_ROWS // NS
OROWS = N // NS
ISQ = float(1.0 / math.sqrt(1.0 + 1e-5))

NCH_HALF = 80
NCH_FULL = 158


def _fill(ref, rows, width, val):
    v = jnp.full((16,), val, jnp.float32)

    def row(i, _):
        for k in range(width // 16):
            ref[i, pl.ds(16 * k, 16)] = v
        return 0

    lax.fori_loop(0, rows, row, 0)


def _make_agg(F, nch, do_gather):
    mesh = plsc.VectorSubcoreMesh(core_axis_name="c", subcore_axis_name="s")
    scratch = [
        pltpu.VMEM((nch, CH), jnp.int32),
        pltpu.VMEM((128, F), jnp.float32),
        pltpu.VMEM_SHARED((ACC_ROWS, F), jnp.float32),
    ]
    if do_gather:
        scratch += [
            pltpu.VMEM((nch, CH), jnp.int32),
            pltpu.VMEM((128, F), jnp.float32),
            pltpu.SemaphoreType.DMA,
            pltpu.SemaphoreType.DMA,
        ]

    def body(table, src_hbm, dst_hbm, out, dst_v, buf0, acc, *rest):
        c = lax.axis_index("c")
        s = lax.axis_index("s")
        wid = c * NS + s
        pltpu.sync_copy(dst_hbm.at[wid], dst_v)
        if do_gather:
            src_v, buf1, sem0, sem1 = rest
            pltpu.sync_copy(src_hbm.at[wid], src_v)
        _fill(buf0, 128, F, 0.0)
        for off in range(0, ZROWS, 128):
            n = min(128, ZROWS - off)
            pltpu.sync_copy(buf0.at[pl.ds(0, n)],
                            acc.at[pl.ds(s * ZROWS + off, n)])
        plsc.subcore_barrier()

        if do_gather:
            def gath(j, buf, sem):
                return pltpu.make_async_copy(table.at[src_v.at[j]], buf, sem)

            gath(0, buf0, sem0).start()
            gath(1, buf1, sem1).start()

            def step(i, _):
                j = 2 * i
                for b in range(2):
                    buf, sem = (buf0, sem0) if b == 0 else (buf1, sem1)
                    jj = j + b
                    gath(jj, buf, sem).wait()
                    pltpu.sync_copy(buf, acc.at[dst_v.at[jj]], add=True)

                    @pl.when(jj + 2 < nch)
                    def _():
                        gath(jj + 2, buf, sem).start()
                return 0

            lax.fori_loop(0, nch // 2, step, 0)
        else:
            _fill(buf0, 128, F, 1.0)

            def step(j, _):
                pltpu.sync_copy(buf0, acc.at[dst_v.at[j]], add=True)
                return 0

            lax.fori_loop(0, nch, step, 0)

        plsc.subcore_barrier()
        pltpu.sync_copy(acc.at[pl.ds(s * OROWS, OROWS)],
                        out.at[c, pl.ds(s * OROWS, OROWS)])

    return pl.kernel(
        body,
        out_type=jax.ShapeDtypeStruct((NC, N, F), jnp.float32),
        mesh=mesh,
        scratch_types=scratch,
    )


_agg128 = _make_agg(128, NCH_HALF, True)
_agg128_full = _make_agg(128, NCH_FULL, True)
_agg64 = _make_agg(64, NCH_HALF, True)
_deg16 = _make_agg(16, NCH_HALF, False)


def _pad_pack(a, slots, pad, shape):
    p = jnp.concatenate([a, jnp.full((slots - E,), pad, jnp.int32)])
    return p.reshape(shape)


_B = 1250
_GRID = (N // _B,)


def _dinv(degp_ref):
    deg = degp_ref[0, :, 0:1] + degp_ref[1, :, 0:1] + 1.0
    return lax.rsqrt(deg)


def _bn(h, g, be):
    return g * (h * ISQ) + be


def _rows_spec(F):
    return pl.BlockSpec((_B, F), lambda i: (i, 0))


def _pair_spec(F):
    return pl.BlockSpec((2, _B, F), lambda i: (0, i, 0))


def _full_spec(shape):
    nd = len(shape)
    return pl.BlockSpec(shape, lambda i: (0,) * nd)


def _stage_a(x, degp, W1):
    def f(x_ref, degp_ref, w_ref, xs1_ref):
        dinv = _dinv(degp_ref)
        xs1_ref[...] = dinv * jnp.dot(x_ref[...], w_ref[...],
                                      preferred_element_type=jnp.float32)

    return pl.pallas_call(
        f,
        grid=_GRID,
        in_specs=[_rows_spec(128), _pair_spec(16), _full_spec((128, 128))],
        out_specs=_rows_spec(128),
        out_shape=jax.ShapeDtypeStruct((N, 128), jnp.float32),
    )(x, degp, W1)


def _stage_b(p1, xs1, degp, b1, g1, be1, W2):
    def f(p_ref, xs_ref, degp_ref, b_ref, g_ref, be_ref, w_ref, out_ref):
        dinv = _dinv(degp_ref)
        pre = dinv * (p_ref[0] + p_ref[1] + xs_ref[...]) + b_ref[...]
        h = _bn(jnp.maximum(pre, 0.0), g_ref[...], be_ref[...])
        out_ref[...] = dinv * jnp.dot(h, w_ref[...],
                                      preferred_element_type=jnp.float32)

    return pl.pallas_call(
        f,
        grid=_GRID,
        in_specs=[_pair_spec(128), _rows_spec(128), _pair_spec(16),
                  _full_spec((1, 128)), _full_spec((1, 128)),
                  _full_spec((1, 128)), _full_spec((128, 256))],
        out_specs=_rows_spec(256),
        out_shape=jax.ShapeDtypeStruct((N, 256), jnp.float32),
    )(p1, xs1, degp, b1, g1, be1, W2)


def _stage_c(a2, xs2, degp, b2, g2, be2, W3):
    def f(a_ref, xs_ref, degp_ref, b_ref, g_ref, be_ref, w_ref, out_ref):
        dinv = _dinv(degp_ref)
        acc = None
        for k in range(2):
            sl = slice(128 * k, 128 * (k + 1))
            pre = dinv * (a_ref[k] + xs_ref[:, sl]) + b_ref[:, sl]
            h = _bn(jnp.maximum(pre, 0.0), g_ref[:, sl], be_ref[:, sl])
            part = jnp.dot(h, w_ref[sl, :], preferred_element_type=jnp.float32)
            acc = part if acc is None else acc + part
        out_ref[...] = dinv * acc

    return pl.pallas_call(
        f,
        grid=_GRID,
        in_specs=[_pair_spec(128), _rows_spec(256), _pair_spec(16),
                  _full_spec((1, 256)), _full_spec((1, 256)),
                  _full_spec((1, 256)), _full_spec((256, 128))],
        out_specs=_rows_spec(128),
        out_shape=jax.ShapeDtypeStruct((N, 128), jnp.float32),
    )(a2, xs2, degp, b2, g2, be2, W3)


def _stage_d(p3, xs3, degp, b3, g3, be3, W4):
    def f(p_ref, xs_ref, degp_ref, b_ref, g_ref, be_ref, w_ref, out_ref):
        dinv = _dinv(degp_ref)
        pre = dinv * (p_ref[0] + p_ref[1] + xs_ref[...]) + b_ref[...]
        h = _bn(jnp.maximum(pre, 0.0), g_ref[...], be_ref[...])
        out_ref[...] = dinv * jnp.dot(h, w_ref[...],
                                      preferred_element_type=jnp.float32)

    return pl.pallas_call(
        f,
        grid=_GRID,
        in_specs=[_pair_spec(128), _rows_spec(128), _pair_spec(16),
                  _full_spec((1, 128)), _full_spec((1, 128)),
                  _full_spec((1, 128)), _full_spec((128, 64))],
        out_specs=_rows_spec(64),
        out_shape=jax.ShapeDtypeStruct((N, 64), jnp.float32),
    )(p3, xs3, degp, b3, g3, be3, W4)


def _stage_e(p4, xs4, degp, b4, g4, be4, Wc, bc):
    def f(p_ref, xs_ref, degp_ref, b_ref, g_ref, be_ref, wc_ref, bc_ref,
          logits_ref, out4_ref, obn4_ref):
        dinv = _dinv(degp_ref)
        pre = dinv * (p_ref[0] + p_ref[1] + xs_ref[...]) + b_ref[...]
        out4 = jnp.maximum(pre, 0.0)
        obn4 = _bn(out4, g_ref[...], be_ref[...])
        out4_ref[...] = out4
        obn4_ref[...] = obn4
        logits_ref[...] = jnp.dot(obn4, wc_ref[...],
                                  preferred_element_type=jnp.float32) + bc_ref[...]

    return pl.pallas_call(
        f,
        grid=_GRID,
        in_specs=[_pair_spec(64), _rows_spec(64), _pair_spec(16),
                  _full_spec((1, 64)), _full_spec((1, 64)),
                  _full_spec((1, 64)), _full_spec((64, 10)),
                  _full_spec((1, 10))],
        out_specs=[_rows_spec(10), _rows_spec(64), _rows_spec(64)],
        out_shape=[jax.ShapeDtypeStruct((N, 10), jnp.float32),
                   jax.ShapeDtypeStruct((N, 64), jnp.float32),
                   jax.ShapeDtypeStruct((N, 64), jnp.float32)],
    )(p4, xs4, degp, b4, g4, be4, Wc, bc)


def kernel(x, edge_index, W1, b1, g1, be1, W2, b2, g2, be2,
           W3, b3, g3, be3, W4, b4, g4, be4, Wc, bc):
    src = edge_index[0].astype(jnp.int32)
    dst = edge_index[1].astype(jnp.int32)

    half_slots = NW * NCH_HALF * CH
    srcW = _pad_pack(src, half_slots, 0, (NW, NCH_HALF, CH))
    dstW = _pad_pack(dst, half_slots, N, (NW, NCH_HALF, CH))

    full_slots = NS * NCH_FULL * CH
    s2 = 2 * src
    src2W = jnp.stack([
        _pad_pack(s2, full_slots, 0, (NS, NCH_FULL, CH)),
        _pad_pack(s2 + 1, full_slots, 0, (NS, NCH_FULL, CH)),
    ]).reshape(NW, NCH_FULL, CH)
    d2 = _pad_pack(dst, full_slots, N, (NS, NCH_FULL, CH))
    dst2W = jnp.stack([d2, d2]).reshape(NW, NCH_FULL, CH)

    ones_tbl = jnp.zeros((8, 16), jnp.float32)
    degp = _deg16(ones_tbl, srcW, dstW)

    b1r, g1r, be1r = b1.reshape(1, -1), g1.reshape(1, -1), be1.reshape(1, -1)
    b2r, g2r, be2r = b2.reshape(1, -1), g2.reshape(1, -1), be2.reshape(1, -1)
    b3r, g3r, be3r = b3.reshape(1, -1), g3.reshape(1, -1), be3.reshape(1, -1)
    b4r, g4r, be4r = b4.reshape(1, -1), g4.reshape(1, -1), be4.reshape(1, -1)
    bcr = bc.reshape(1, -1)

    xs1 = _stage_a(x, degp, W1)
    p1 = _agg128(xs1, srcW, dstW)
    xs2 = _stage_b(p1, xs1, degp, b1r, g1r, be1r, W2)
    a2 = _agg128_full(xs2.reshape(2 * N, 128), src2W, dst2W)
    xs3 = _stage_c(a2, xs2, degp, b2r, g2r, be2r, W3)
    p3 = _agg128(xs3, srcW, dstW)
    xs4 = _stage_d(p3, xs3, degp, b3r, g3r, be3r, W4)
    p4 = _agg64(xs4, srcW, dstW)
    logits, out4, obn4 = _stage_e(p4, xs4, degp, b4r, g4r, be4r, Wc, bcr)
    return (logits, out4, obn4)


# speedup vs baseline: 6.6684x; 6.6684x over previous
import math

import jax
import jax.numpy as jnp
from jax import lax
from jax.experimental import pallas as pl
from jax.experimental.pallas import tpu as pltpu
from jax.experimental.pallas import tpu_sc as plsc

N = 10000
E = 320000
NC = 2
NS = 16
NW = NC * NS
CH = 128
ACC_ROWS = 10112
ZROWS = ACC_ROWS // NS
OROWS = ACC_ROWS // NS
ISQ = float(1.0 / math.sqrt(1.0 + 1e-5))

NCHP = 40
NPH_HALF = 2
NPH_FULL = 4


def _fill(ref, rows, width, val):
    v = jnp.full((16,), val, jnp.float32)

    def row(i, _):
        for k in range(width // 16):
            ref[i, pl.ds(16 * k, 16)] = v
        return 0

    lax.fori_loop(0, rows, row, 0)


def _make_agg(F, nph, do_gather):
    mesh = plsc.VectorSubcoreMesh(core_axis_name="c", subcore_axis_name="s")
    scratch = [
        pltpu.VMEM((NCHP, CH), jnp.int32),
        pltpu.VMEM((128, F), jnp.float32),
        pltpu.VMEM_SHARED((ACC_ROWS, F), jnp.float32),
    ]
    if do_gather:
        scratch += [
            pltpu.VMEM((NCHP, CH), jnp.int32),
            pltpu.VMEM((128, F), jnp.float32),
            pltpu.SemaphoreType.DMA,
            pltpu.SemaphoreType.DMA,
        ]

    def body(table, src_hbm, dst_hbm, out, dst_v, buf0, acc, *rest):
        c = lax.axis_index("c")
        s = lax.axis_index("s")
        wid = c * NS + s
        if do_gather:
            src_v, buf1, sem0, sem1 = rest
        _fill(buf0, 128, F, 0.0)
        for off in range(0, ZROWS, 128):
            n = min(128, ZROWS - off)
            pltpu.sync_copy(buf0.at[pl.ds(0, n)],
                            acc.at[pl.ds(s * ZROWS + off, n)])
        if not do_gather:
            _fill(buf0, 128, F, 1.0)
        plsc.subcore_barrier()

        for ph in range(nph):
            pltpu.sync_copy(dst_hbm.at[wid, ph], dst_v)
            if do_gather:
                pltpu.sync_copy(src_hbm.at[wid, ph], src_v)

                def gath(j, buf, sem):
                    return pltpu.make_async_copy(
                        table.at[src_v.at[j]], buf, sem)

                gath(0, buf0, sem0).start()
                gath(1, buf1, sem1).start()

                def step(i, _):
                    j = 2 * i
                    for b in range(2):
                        buf, sem = (buf0, sem0) if b == 0 else (buf1, sem1)
                        jj = j + b
                        gath(jj, buf, sem).wait()
                        pltpu.sync_copy(buf, acc.at[dst_v.at[jj]], add=True)

                        @pl.when(jj + 2 < NCHP)
                        def _():
                            gath(jj + 2, buf, sem).start()
                    return 0

                lax.fori_loop(0, NCHP // 2, step, 0)
            else:
                def step(j, _):
                    pltpu.sync_copy(buf0, acc.at[dst_v.at[j]], add=True)
                    return 0

                lax.fori_loop(0, NCHP, step, 0)

        plsc.subcore_barrier()
        pltpu.sync_copy(acc.at[pl.ds(s * OROWS, OROWS)],
                        out.at[c, pl.ds(s * OROWS, OROWS)])

    return pl.kernel(
        body,
        out_type=jax.ShapeDtypeStruct((NC, ACC_ROWS, F), jnp.float32),
        mesh=mesh,
        scratch_types=scratch,
    )


_agg128 = _make_agg(128, NPH_HALF, True)
_agg128_full = _make_agg(128, NPH_FULL, True)
_deg128 = _make_agg(128, NPH_HALF, False)


def _pad_pack(a, slots, pad, shape):
    p = jnp.concatenate([a, jnp.full((slots - E,), pad, jnp.int32)])
    return p.reshape(shape)


_B = 1000
_GRID = (N // _B,)


def _dinv(degp_ref):
    deg = degp_ref[0, :, 0:1] + degp_ref[1, :, 0:1] + 1.0
    return lax.rsqrt(deg)


def _bn(h, g, be):
    return g * (h * ISQ) + be


def _rows_spec(F):
    return pl.BlockSpec((_B, F), lambda i: (i, 0))


def _pair_spec(F):
    return pl.BlockSpec((2, _B, F), lambda i: (0, i, 0))


def _full_spec(shape):
    nd = len(shape)
    return pl.BlockSpec(shape, lambda i: (0,) * nd)


def _stage_a(x, degp, W1):
    def f(x_ref, degp_ref, w_ref, xs1_ref):
        dinv = _dinv(degp_ref)
        xs1_ref[...] = dinv * jnp.dot(x_ref[...], w_ref[...],
                                      preferred_element_type=jnp.float32)

    return pl.pallas_call(
        f,
        grid=_GRID,
        in_specs=[_rows_spec(128), _pair_spec(128), _full_spec((128, 128))],
        out_specs=_rows_spec(128),
        out_shape=jax.ShapeDtypeStruct((N, 128), jnp.float32),
    )(x, degp, W1)


def _stage_b(p1, xs1, degp, b1, g1, be1, W2):
    def f(p_ref, xs_ref, degp_ref, b_ref, g_ref, be_ref, w_ref, out_ref):
        dinv = _dinv(degp_ref)
        pre = dinv * (p_ref[0] + p_ref[1] + xs_ref[...]) + b_ref[...]
        h = _bn(jnp.maximum(pre, 0.0), g_ref[...], be_ref[...])
        out_ref[...] = dinv * jnp.dot(h, w_ref[...],
                                      preferred_element_type=jnp.float32)

    return pl.pallas_call(
        f,
        grid=_GRID,
        in_specs=[_pair_spec(128), _rows_spec(128), _pair_spec(128),
                  _full_spec((1, 128)), _full_spec((1, 128)),
                  _full_spec((1, 128)), _full_spec((128, 256))],
        out_specs=_rows_spec(256),
        out_shape=jax.ShapeDtypeStruct((N, 256), jnp.float32),
    )(p1, xs1, degp, b1, g1, be1, W2)


def _stage_c(a2, xs2, degp, b2, g2, be2, W3):
    def f(a_ref, xs_ref, degp_ref, b_ref, g_ref, be_ref, w_ref, out_ref):
        dinv = _dinv(degp_ref)
        acc = None
        for k in range(2):
            sl = slice(128 * k, 128 * (k + 1))
            pre = dinv * (a_ref[k] + xs_ref[:, sl]) + b_ref[:, sl]
            h = _bn(jnp.maximum(pre, 0.0), g_ref[:, sl], be_ref[:, sl])
            part = jnp.dot(h, w_ref[sl, :], preferred_element_type=jnp.float32)
            acc = part if acc is None else acc + part
        out_ref[...] = dinv * acc

    return pl.pallas_call(
        f,
        grid=_GRID,
        in_specs=[_pair_spec(128), _rows_spec(256), _pair_spec(128),
                  _full_spec((1, 256)), _full_spec((1, 256)),
                  _full_spec((1, 256)), _full_spec((256, 128))],
        out_specs=_rows_spec(128),
        out_shape=jax.ShapeDtypeStruct((N, 128), jnp.float32),
    )(a2, xs2, degp, b2, g2, be2, W3)


def _stage_d(p3, xs3, degp, b3, g3, be3, W4):
    def f(p_ref, xs_ref, degp_ref, b_ref, g_ref, be_ref, w_ref, out_ref):
        dinv = _dinv(degp_ref)
        pre = dinv * (p_ref[0] + p_ref[1] + xs_ref[...]) + b_ref[...]
        h = _bn(jnp.maximum(pre, 0.0), g_ref[...], be_ref[...])
        xs4 = dinv * jnp.dot(h, w_ref[...],
                             preferred_element_type=jnp.float32)
        out_ref[...] = jnp.concatenate(
            [xs4, jnp.zeros((_B, 64), jnp.float32)], axis=1)

    return pl.pallas_call(
        f,
        grid=_GRID,
        in_specs=[_pair_spec(128), _rows_spec(128), _pair_spec(128),
                  _full_spec((1, 128)), _full_spec((1, 128)),
                  _full_spec((1, 128)), _full_spec((128, 64))],
        out_specs=_rows_spec(128),
        out_shape=jax.ShapeDtypeStruct((N, 128), jnp.float32),
    )(p3, xs3, degp, b3, g3, be3, W4)


def _stage_e(p4, xs4, degp, b4, g4, be4, Wc, bc):
    def f(p_ref, xs_ref, degp_ref, b_ref, g_ref, be_ref, wc_ref, bc_ref,
          logits_ref, out4_ref, obn4_ref):
        dinv = _dinv(degp_ref)
        pre = (dinv * (p_ref[0, :, 0:64] + p_ref[1, :, 0:64]
                       + xs_ref[:, 0:64]) + b_ref[...])
        out4 = jnp.maximum(pre, 0.0)
        obn4 = _bn(out4, g_ref[...], be_ref[...])
        out4_ref[...] = out4
        obn4_ref[...] = obn4
        logits_ref[...] = jnp.dot(obn4, wc_ref[...],
                                  preferred_element_type=jnp.float32) + bc_ref[...]

    return pl.pallas_call(
        f,
        grid=_GRID,
        in_specs=[_pair_spec(128), _rows_spec(128), _pair_spec(128),
                  _full_spec((1, 64)), _full_spec((1, 64)),
                  _full_spec((1, 64)), _full_spec((64, 10)),
                  _full_spec((1, 10))],
        out_specs=[_rows_spec(10), _rows_spec(64), _rows_spec(64)],
        out_shape=[jax.ShapeDtypeStruct((N, 10), jnp.float32),
                   jax.ShapeDtypeStruct((N, 64), jnp.float32),
                   jax.ShapeDtypeStruct((N, 64), jnp.float32)],
    )(p4, xs4, degp, b4, g4, be4, Wc, bc)


def kernel(x, edge_index, W1, b1, g1, be1, W2, b2, g2, be2,
           W3, b3, g3, be3, W4, b4, g4, be4, Wc, bc):
    src = edge_index[0].astype(jnp.int32)
    dst = edge_index[1].astype(jnp.int32)

    half_shape = (NW, NPH_HALF, NCHP, CH)
    half_slots = NW * NPH_HALF * NCHP * CH
    srcW = _pad_pack(src, half_slots, 0, half_shape)
    dstW = _pad_pack(dst, half_slots, N, half_shape)

    core_shape = (NS, NPH_FULL, NCHP, CH)
    full_slots = NS * NPH_FULL * NCHP * CH
    s2 = 2 * src
    src2W = jnp.stack([
        _pad_pack(s2, full_slots, 0, core_shape),
        _pad_pack(s2 + 1, full_slots, 0, core_shape),
    ]).reshape((NW,) + core_shape[1:])
    d2 = _pad_pack(dst, full_slots, N, core_shape)
    dst2W = jnp.stack([d2, d2]).reshape((NW,) + core_shape[1:])

    ones_tbl = jnp.zeros((8, 128), jnp.float32)
    degp = _deg128(ones_tbl, srcW, dstW)

    b1r, g1r, be1r = b1.reshape(1, -1), g1.reshape(1, -1), be1.reshape(1, -1)
    b2r, g2r, be2r = b2.reshape(1, -1), g2.reshape(1, -1), be2.reshape(1, -1)
    b3r, g3r, be3r = b3.reshape(1, -1), g3.reshape(1, -1), be3.reshape(1, -1)
    b4r, g4r, be4r = b4.reshape(1, -1), g4.reshape(1, -1), be4.reshape(1, -1)
    bcr = bc.reshape(1, -1)

    xs1 = _stage_a(x, degp, W1)
    p1 = _agg128(xs1, srcW, dstW)
    xs2 = _stage_b(p1, xs1, degp, b1r, g1r, be1r, W2)
    a2 = _agg128_full(xs2.reshape(2 * N, 128), src2W, dst2W)
    xs3 = _stage_c(a2, xs2, degp, b2r, g2r, be2r, W3)
    p3 = _agg128(xs3, srcW, dstW)
    xs4 = _stage_d(p3, xs3, degp, b3r, g3r, be3r, W4)
    p4 = _agg128(xs4, srcW, dstW)
    logits, out4, obn4 = _stage_e(p4, xs4, degp, b4r, g4r, be4r, Wc, bcr)
    return (logits, out4, obn4)


# speedup vs baseline: 6.6943x; 1.0039x over previous
import math

import jax
import jax.numpy as jnp
from jax import lax
from jax.experimental import pallas as pl
from jax.experimental.pallas import tpu as pltpu
from jax.experimental.pallas import tpu_sc as plsc

N = 10000
E = 320000
NC = 2
NS = 16
NW = NC * NS
CH = 64
NBUF = 4
ACC_ROWS = 10112
ZROWS = ACC_ROWS // NS
OROWS = ACC_ROWS // NS
ISQ = float(1.0 / math.sqrt(1.0 + 1e-5))

NCHP = 40
NPH_HALF = 4
NPH_FULL = 8


def _fill(ref, rows, width, val):
    v = jnp.full((16,), val, jnp.float32)

    def row(i, _):
        for k in range(width // 16):
            ref[i, pl.ds(16 * k, 16)] = v
        return 0

    lax.fori_loop(0, rows, row, 0)


def _make_agg(F, nph, do_gather):
    mesh = plsc.VectorSubcoreMesh(core_axis_name="c", subcore_axis_name="s")
    scratch = [
        pltpu.VMEM((NCHP, CH), jnp.int32),
        pltpu.VMEM((CH, F), jnp.float32),
        pltpu.VMEM_SHARED((ACC_ROWS, F), jnp.float32),
    ]
    if do_gather:
        scratch += [pltpu.VMEM((NCHP, CH), jnp.int32)]
        scratch += [pltpu.VMEM((CH, F), jnp.float32)] * (NBUF - 1)
        scratch += [pltpu.SemaphoreType.DMA] * NBUF

    def body(table, src_hbm, dst_hbm, out, dst_v, buf0, acc, *rest):
        c = lax.axis_index("c")
        s = lax.axis_index("s")
        wid = c * NS + s
        if do_gather:
            src_v = rest[0]
            bufs = (buf0,) + rest[1:NBUF]
            sems = rest[NBUF:]
        _fill(buf0, CH, F, 0.0)
        for off in range(0, ZROWS, CH):
            n = min(CH, ZROWS - off)
            pltpu.sync_copy(buf0.at[pl.ds(0, n)],
                            acc.at[pl.ds(s * ZROWS + off, n)])
        if not do_gather:
            _fill(buf0, CH, F, 1.0)
        plsc.subcore_barrier()

        for ph in range(nph):
            pltpu.sync_copy(dst_hbm.at[wid, ph], dst_v)
            if do_gather:
                pltpu.sync_copy(src_hbm.at[wid, ph], src_v)

                def gath(j, buf, sem):
                    return pltpu.make_async_copy(
                        table.at[src_v.at[j]], buf, sem)

                for b in range(NBUF):
                    gath(b, bufs[b], sems[b]).start()

                def step(i, _):
                    j = NBUF * i
                    for b in range(NBUF):
                        jj = j + b
                        gath(jj, bufs[b], sems[b]).wait()
                        pltpu.sync_copy(bufs[b], acc.at[dst_v.at[jj]],
                                        add=True)

                        @pl.when(jj + NBUF < NCHP)
                        def _():
                            gath(jj + NBUF, bufs[b], sems[b]).start()
                    return 0

                lax.fori_loop(0, NCHP // NBUF, step, 0)
            else:
                def step(j, _):
                    pltpu.sync_copy(buf0, acc.at[dst_v.at[j]], add=True)
                    return 0

                lax.fori_loop(0, NCHP, step, 0)

        plsc.subcore_barrier()
        pltpu.sync_copy(acc.at[pl.ds(s * OROWS, OROWS)],
                        out.at[c, pl.ds(s * OROWS, OROWS)])

    return pl.kernel(
        body,
        out_type=jax.ShapeDtypeStruct((NC, ACC_ROWS, F), jnp.float32),
        mesh=mesh,
        scratch_types=scratch,
    )


_agg128 = _make_agg(128, NPH_HALF, True)
_agg128_full = _make_agg(128, NPH_FULL, True)
_deg128 = _make_agg(128, NPH_HALF, False)


def _pad_pack(a, slots, pad, shape):
    p = jnp.concatenate([a, jnp.full((slots - E,), pad, jnp.int32)])
    return p.reshape(shape)


_B = 1000
_GRID = (N // _B,)


def _dinv(degp_ref):
    deg = degp_ref[0, :, 0:1] + degp_ref[1, :, 0:1] + 1.0
    return lax.rsqrt(deg)


def _bn(h, g, be):
    return g * (h * ISQ) + be


def _rows_spec(F):
    return pl.BlockSpec((_B, F), lambda i: (i, 0))


def _pair_spec(F):
    return pl.BlockSpec((2, _B, F), lambda i: (0, i, 0))


def _full_spec(shape):
    nd = len(shape)
    return pl.BlockSpec(shape, lambda i: (0,) * nd)


def _stage_a(x, degp, W1):
    def f(x_ref, degp_ref, w_ref, xs1_ref):
        dinv = _dinv(degp_ref)
        xs1_ref[...] = dinv * jnp.dot(x_ref[...], w_ref[...],
                                      preferred_element_type=jnp.float32)

    return pl.pallas_call(
        f,
        grid=_GRID,
        in_specs=[_rows_spec(128), _pair_spec(128), _full_spec((128, 128))],
        out_specs=_rows_spec(128),
        out_shape=jax.ShapeDtypeStruct((N, 128), jnp.float32),
    )(x, degp, W1)


def _stage_b(p1, xs1, degp, b1, g1, be1, W2):
    def f(p_ref, xs_ref, degp_ref, b_ref, g_ref, be_ref, w_ref, out_ref):
        dinv = _dinv(degp_ref)
        pre = dinv * (p_ref[0] + p_ref[1] + xs_ref[...]) + b_ref[...]
        h = _bn(jnp.maximum(pre, 0.0), g_ref[...], be_ref[...])
        out_ref[...] = dinv * jnp.dot(h, w_ref[...],
                                      preferred_element_type=jnp.float32)

    return pl.pallas_call(
        f,
        grid=_GRID,
        in_specs=[_pair_spec(128), _rows_spec(128), _pair_spec(128),
                  _full_spec((1, 128)), _full_spec((1, 128)),
                  _full_spec((1, 128)), _full_spec((128, 256))],
        out_specs=_rows_spec(256),
        out_shape=jax.ShapeDtypeStruct((N, 256), jnp.float32),
    )(p1, xs1, degp, b1, g1, be1, W2)


def _stage_c(a2, xs2, degp, b2, g2, be2, W3):
    def f(a_ref, xs_ref, degp_ref, b_ref, g_ref, be_ref, w_ref, out_ref):
        dinv = _dinv(degp_ref)
        acc = None
        for k in range(2):
            sl = slice(128 * k, 128 * (k + 1))
            pre = dinv * (a_ref[k] + xs_ref[:, sl]) + b_ref[:, sl]
            h = _bn(jnp.maximum(pre, 0.0), g_ref[:, sl], be_ref[:, sl])
            part = jnp.dot(h, w_ref[sl, :], preferred_element_type=jnp.float32)
            acc = part if acc is None else acc + part
        out_ref[...] = dinv * acc

    return pl.pallas_call(
        f,
        grid=_GRID,
        in_specs=[_pair_spec(128), _rows_spec(256), _pair_spec(128),
                  _full_spec((1, 256)), _full_spec((1, 256)),
                  _full_spec((1, 256)), _full_spec((256, 128))],
        out_specs=_rows_spec(128),
        out_shape=jax.ShapeDtypeStruct((N, 128), jnp.float32),
    )(a2, xs2, degp, b2, g2, be2, W3)


def _stage_d(p3, xs3, degp, b3, g3, be3, W4):
    def f(p_ref, xs_ref, degp_ref, b_ref, g_ref, be_ref, w_ref, out_ref):
        dinv = _dinv(degp_ref)
        pre = dinv * (p_ref[0] + p_ref[1] + xs_ref[...]) + b_ref[...]
        h = _bn(jnp.maximum(pre, 0.0), g_ref[...], be_ref[...])
        xs4 = dinv * jnp.dot(h, w_ref[...],
                             preferred_element_type=jnp.float32)
        out_ref[...] = jnp.concatenate(
            [xs4, jnp.zeros((_B, 64), jnp.float32)], axis=1)

    return pl.pallas_call(
        f,
        grid=_GRID,
        in_specs=[_pair_spec(128), _rows_spec(128), _pair_spec(128),
                  _full_spec((1, 128)), _full_spec((1, 128)),
                  _full_spec((1, 128)), _full_spec((128, 64))],
        out_specs=_rows_spec(128),
        out_shape=jax.ShapeDtypeStruct((N, 128), jnp.float32),
    )(p3, xs3, degp, b3, g3, be3, W4)


def _stage_e(p4, xs4, degp, b4, g4, be4, Wc, bc):
    def f(p_ref, xs_ref, degp_ref, b_ref, g_ref, be_ref, wc_ref, bc_ref,
          logits_ref, out4_ref, obn4_ref):
        dinv = _dinv(degp_ref)
        pre = (dinv * (p_ref[0, :, 0:64] + p_ref[1, :, 0:64]
                       + xs_ref[:, 0:64]) + b_ref[...])
        out4 = jnp.maximum(pre, 0.0)
        obn4 = _bn(out4, g_ref[...], be_ref[...])
        out4_ref[...] = out4
        obn4_ref[...] = obn4
        logits_ref[...] = jnp.dot(obn4, wc_ref[...],
                                  preferred_element_type=jnp.float32) + bc_ref[...]

    return pl.pallas_call(
        f,
        grid=_GRID,
        in_specs=[_pair_spec(128), _rows_spec(128), _pair_spec(128),
                  _full_spec((1, 64)), _full_spec((1, 64)),
                  _full_spec((1, 64)), _full_spec((64, 10)),
                  _full_spec((1, 10))],
        out_specs=[_rows_spec(10), _rows_spec(64), _rows_spec(64)],
        out_shape=[jax.ShapeDtypeStruct((N, 10), jnp.float32),
                   jax.ShapeDtypeStruct((N, 64), jnp.float32),
                   jax.ShapeDtypeStruct((N, 64), jnp.float32)],
    )(p4, xs4, degp, b4, g4, be4, Wc, bc)


def kernel(x, edge_index, W1, b1, g1, be1, W2, b2, g2, be2,
           W3, b3, g3, be3, W4, b4, g4, be4, Wc, bc):
    src = edge_index[0].astype(jnp.int32)
    dst = edge_index[1].astype(jnp.int32)

    half_shape = (NW, NPH_HALF, NCHP, CH)
    half_slots = NW * NPH_HALF * NCHP * CH
    srcW = _pad_pack(src, half_slots, 0, half_shape)
    dstW = _pad_pack(dst, half_slots, N, half_shape)

    core_shape = (NS, NPH_FULL, NCHP, CH)
    full_slots = NS * NPH_FULL * NCHP * CH
    s2 = 2 * src
    src2W = jnp.stack([
        _pad_pack(s2, full_slots, 0, core_shape),
        _pad_pack(s2 + 1, full_slots, 0, core_shape),
    ]).reshape((NW,) + core_shape[1:])
    d2 = _pad_pack(dst, full_slots, N, core_shape)
    dst2W = jnp.stack([d2, d2]).reshape((NW,) + core_shape[1:])

    ones_tbl = jnp.zeros((8, 128), jnp.float32)
    degp = _deg128(ones_tbl, srcW, dstW)

    b1r, g1r, be1r = b1.reshape(1, -1), g1.reshape(1, -1), be1.reshape(1, -1)
    b2r, g2r, be2r = b2.reshape(1, -1), g2.reshape(1, -1), be2.reshape(1, -1)
    b3r, g3r, be3r = b3.reshape(1, -1), g3.reshape(1, -1), be3.reshape(1, -1)
    b4r, g4r, be4r = b4.reshape(1, -1), g4.reshape(1, -1), be4.reshape(1, -1)
    bcr = bc.reshape(1, -1)

    xs1 = _stage_a(x, degp, W1)
    p1 = _agg128(xs1, srcW, dstW)
    xs2 = _stage_b(p1, xs1, degp, b1r, g1r, be1r, W2)
    a2 = _agg128_full(xs2.reshape(2 * N, 128), src2W, dst2W)
    xs3 = _stage_c(a2, xs2, degp, b2r, g2r, be2r, W3)
    p3 = _agg128(xs3, srcW, dstW)
    xs4 = _stage_d(p3, xs3, degp, b3r, g3r, be3r, W4)
    p4 = _agg128(xs4, srcW, dstW)
    logits, out4, obn4 = _stage_e(p4, xs4, degp, b4r, g4r, be4r, Wc, bcr)
    return (logits, out4, obn4)


# speedup vs baseline: 6.7240x; 1.0044x over previous
import math

import jax
import jax.numpy as jnp
from jax import lax
from jax.experimental import pallas as pl
from jax.experimental.pallas import tpu as pltpu
from jax.experimental.pallas import tpu_sc as plsc

N = 10000
E = 320000
NC = 2
NS = 16
NW = NC * NS
CH = 64
NBUF = 4
ACC_ROWS = 10112
ZROWS = ACC_ROWS // NS
OROWS = ACC_ROWS // NS
ISQ = float(1.0 / math.sqrt(1.0 + 1e-5))

NCHP = 40
NPH_HALF = 4

SLAB = 5120
SROWS = SLAB // NS
OFF1 = N - SLAB
CH2 = 16
NCHP2 = 12
NPH2 = 108


def _fill(ref, rows, width, val):
    v = jnp.full((16,), val, jnp.float32)

    def row(i, _):
        for k in range(width // 16):
            ref[i, pl.ds(16 * k, 16)] = v
        return 0

    lax.fori_loop(0, rows, row, 0)


def _make_agg(F, nph, do_gather):
    mesh = plsc.VectorSubcoreMesh(core_axis_name="c", subcore_axis_name="s")
    scratch = [
        pltpu.VMEM((NCHP, CH), jnp.int32),
        pltpu.VMEM((CH, F), jnp.float32),
        pltpu.VMEM_SHARED((ACC_ROWS, F), jnp.float32),
    ]
    if do_gather:
        scratch += [pltpu.VMEM((NCHP, CH), jnp.int32)]
        scratch += [pltpu.VMEM((CH, F), jnp.float32)] * (NBUF - 1)
        scratch += [pltpu.SemaphoreType.DMA] * NBUF

    def body(table, src_hbm, dst_hbm, out, dst_v, buf0, acc, *rest):
        c = lax.axis_index("c")
        s = lax.axis_index("s")
        wid = c * NS + s
        if do_gather:
            src_v = rest[0]
            bufs = (buf0,) + rest[1:NBUF]
            sems = rest[NBUF:]
        _fill(buf0, CH, F, 0.0)
        for off in range(0, ZROWS, CH):
            n = min(CH, ZROWS - off)
            pltpu.sync_copy(buf0.at[pl.ds(0, n)],
                            acc.at[pl.ds(s * ZROWS + off, n)])
        if not do_gather:
            _fill(buf0, CH, F, 1.0)
        plsc.subcore_barrier()

        for ph in range(nph):
            pltpu.sync_copy(dst_hbm.at[wid, ph], dst_v)
            if do_gather:
                pltpu.sync_copy(src_hbm.at[wid, ph], src_v)

                def gath(j, buf, sem):
                    return pltpu.make_async_copy(
                        table.at[src_v.at[j]], buf, sem)

                for b in range(NBUF):
                    gath(b, bufs[b], sems[b]).start()

                def step(i, _):
                    j = NBUF * i
                    for b in range(NBUF):
                        jj = j + b
                        gath(jj, bufs[b], sems[b]).wait()
                        pltpu.sync_copy(bufs[b], acc.at[dst_v.at[jj]],
                                        add=True)

                        @pl.when(jj + NBUF < NCHP)
                        def _():
                            gath(jj + NBUF, bufs[b], sems[b]).start()
                    return 0

                lax.fori_loop(0, NCHP // NBUF, step, 0)
            else:
                def step(j, _):
                    pltpu.sync_copy(buf0, acc.at[dst_v.at[j]], add=True)
                    return 0

                lax.fori_loop(0, NCHP, step, 0)

        plsc.subcore_barrier()
        pltpu.sync_copy(acc.at[pl.ds(s * OROWS, OROWS)],
                        out.at[c, pl.ds(s * OROWS, OROWS)])

    return pl.kernel(
        body,
        out_type=jax.ShapeDtypeStruct((NC, ACC_ROWS, F), jnp.float32),
        mesh=mesh,
        scratch_types=scratch,
    )


def _make_agg_spmem():
    mesh = plsc.VectorSubcoreMesh(core_axis_name="c", subcore_axis_name="s")
    F = 128
    scratch = [
        pltpu.VMEM((NCHP2, CH2), jnp.int32),
        pltpu.VMEM((CH2, F), jnp.float32),
        pltpu.VMEM_SHARED((SLAB, F), jnp.float32),
        pltpu.VMEM_SHARED((ACC_ROWS, F), jnp.float32),
        pltpu.VMEM((NCHP2, CH2), jnp.int32),
        pltpu.VMEM((CH2, F), jnp.float32),
        pltpu.SemaphoreType.DMA,
        pltpu.SemaphoreType.DMA,
    ]

    def body(table, src_hbm, dst_hbm, out,
             dst_v, buf0, tab_s, acc, src_v, buf1, sem0, sem1):
        c = lax.axis_index("c")
        s = lax.axis_index("s")
        wid = c * NS + s
        off = c * OFF1
        pltpu.sync_copy(table.at[pl.ds(off + s * SROWS, SROWS)],
                        tab_s.at[pl.ds(s * SROWS, SROWS)])
        _fill(buf0, CH2, F, 0.0)
        for o in range(0, ZROWS, CH2):
            n = min(CH2, ZROWS - o)
            pltpu.sync_copy(buf0.at[pl.ds(0, n)],
                            acc.at[pl.ds(s * ZROWS + o, n)])
        plsc.subcore_barrier()

        def phase(ph, _):
            pltpu.sync_copy(dst_hbm.at[wid, ph], dst_v)
            pltpu.sync_copy(src_hbm.at[wid, ph], src_v)

            def gath(j, buf, sem):
                return pltpu.make_async_copy(tab_s.at[src_v.at[j]], buf, sem)

            gath(0, buf0, sem0).start()
            gath(1, buf1, sem1).start()

            def step(i, _):
                j = 2 * i
                for b in range(2):
                    buf, sem = (buf0, sem0) if b == 0 else (buf1, sem1)
                    jj = j + b
                    gath(jj, buf, sem).wait()
                    pltpu.sync_copy(buf, acc.at[dst_v.at[jj]], add=True)

                    @pl.when(jj + 2 < NCHP2)
                    def _():
                        gath(jj + 2, buf, sem).start()
                return 0

            lax.fori_loop(0, NCHP2 // 2, step, 0)
            return 0

        lax.fori_loop(0, NPH2, phase, 0)

        plsc.subcore_barrier()
        pltpu.sync_copy(acc.at[pl.ds(s * OROWS, OROWS)],
                        out.at[c, pl.ds(s * OROWS, OROWS)])

    return pl.kernel(
        body,
        out_type=jax.ShapeDtypeStruct((NC, ACC_ROWS, 128), jnp.float32),
        mesh=mesh,
        scratch_types=scratch,
    )


_deg128 = _make_agg(128, NPH_HALF, False)
_aggS = _make_agg_spmem()


def _pad_pack(a, slots, pad, shape):
    p = jnp.concatenate([a, jnp.full((slots - E,), pad, jnp.int32)])
    return p.reshape(shape)


_B = 1000
_GRID = (N // _B,)


def _dinv(degp_ref):
    deg = degp_ref[0, :, 0:1] + degp_ref[1, :, 0:1] + 1.0
    return lax.rsqrt(deg)


def _bn(h, g, be):
    return g * (h * ISQ) + be


def _rows_spec(F):
    return pl.BlockSpec((_B, F), lambda i: (i, 0))


def _pair_spec(F):
    return pl.BlockSpec((2, _B, F), lambda i: (0, i, 0))


def _full_spec(shape):
    nd = len(shape)
    return pl.BlockSpec(shape, lambda i: (0,) * nd)


def _stage_a(x, degp, W1):
    def f(x_ref, degp_ref, w_ref, xs1_ref):
        dinv = _dinv(degp_ref)
        xs1_ref[...] = dinv * jnp.dot(x_ref[...], w_ref[...],
                                      preferred_element_type=jnp.float32)

    return pl.pallas_call(
        f,
        grid=_GRID,
        in_specs=[_rows_spec(128), _pair_spec(128), _full_spec((128, 128))],
        out_specs=_rows_spec(128),
        out_shape=jax.ShapeDtypeStruct((N, 128), jnp.float32),
    )(x, degp, W1)


def _stage_b(p1, xs1, degp, b1, g1, be1, W2):
    def f(p_ref, xs_ref, degp_ref, b_ref, g_ref, be_ref, w_ref,
          oa_ref, ob_ref):
        dinv = _dinv(degp_ref)
        pre = dinv * (p_ref[0] + p_ref[1] + xs_ref[...]) + b_ref[...]
        h = _bn(jnp.maximum(pre, 0.0), g_ref[...], be_ref[...])
        xs2 = dinv * jnp.dot(h, w_ref[...],
                             preferred_element_type=jnp.float32)
        oa_ref[...] = xs2[:, 0:128]
        ob_ref[...] = xs2[:, 128:256]

    return pl.pallas_call(
        f,
        grid=_GRID,
        in_specs=[_pair_spec(128), _rows_spec(128), _pair_spec(128),
                  _full_spec((1, 128)), _full_spec((1, 128)),
                  _full_spec((1, 128)), _full_spec((128, 256))],
        out_specs=[_rows_spec(128), _rows_spec(128)],
        out_shape=[jax.ShapeDtypeStruct((N, 128), jnp.float32),
                   jax.ShapeDtypeStruct((N, 128), jnp.float32)],
    )(p1, xs1, degp, b1, g1, be1, W2)


def _stage_c(a2a, a2b, xs2a, xs2b, degp, b2, g2, be2, W3):
    def f(aa_ref, ab_ref, xa_ref, xb_ref, degp_ref, b_ref, g_ref, be_ref,
          w_ref, out_ref):
        dinv = _dinv(degp_ref)
        acc = None
        for k, (a_ref, x_ref) in enumerate(((aa_ref, xa_ref),
                                            (ab_ref, xb_ref))):
            sl = slice(128 * k, 128 * (k + 1))
            pre = dinv * (a_ref[0] + a_ref[1] + x_ref[...]) + b_ref[:, sl]
            h = _bn(jnp.maximum(pre, 0.0), g_ref[:, sl], be_ref[:, sl])
            part = jnp.dot(h, w_ref[sl, :], preferred_element_type=jnp.float32)
            acc = part if acc is None else acc + part
        out_ref[...] = dinv * acc

    return pl.pallas_call(
        f,
        grid=_GRID,
        in_specs=[_pair_spec(128), _pair_spec(128), _rows_spec(128),
                  _rows_spec(128), _pair_spec(128),
                  _full_spec((1, 256)), _full_spec((1, 256)),
                  _full_spec((1, 256)), _full_spec((256, 128))],
        out_specs=_rows_spec(128),
        out_shape=jax.ShapeDtypeStruct((N, 128), jnp.float32),
    )(a2a, a2b, xs2a, xs2b, degp, b2, g2, be2, W3)


def _stage_d(p3, xs3, degp, b3, g3, be3, W4):
    def f(p_ref, xs_ref, degp_ref, b_ref, g_ref, be_ref, w_ref, out_ref):
        dinv = _dinv(degp_ref)
        pre = dinv * (p_ref[0] + p_ref[1] + xs_ref[...]) + b_ref[...]
        h = _bn(jnp.maximum(pre, 0.0), g_ref[...], be_ref[...])
        xs4 = dinv * jnp.dot(h, w_ref[...],
                             preferred_element_type=jnp.float32)
        out_ref[...] = jnp.concatenate(
            [xs4, jnp.zeros((_B, 64), jnp.float32)], axis=1)

    return pl.pallas_call(
        f,
        grid=_GRID,
        in_specs=[_pair_spec(128), _rows_spec(128), _pair_spec(128),
                  _full_spec((1, 128)), _full_spec((1, 128)),
                  _full_spec((1, 128)), _full_spec((128, 64))],
        out_specs=_rows_spec(128),
        out_shape=jax.ShapeDtypeStruct((N, 128), jnp.float32),
    )(p3, xs3, degp, b3, g3, be3, W4)


def _stage_e(p4, xs4, degp, b4, g4, be4, Wc, bc):
    def f(p_ref, xs_ref, degp_ref, b_ref, g_ref, be_ref, wc_ref, bc_ref,
          logits_ref, out4_ref, obn4_ref):
        dinv = _dinv(degp_ref)
        pre = (dinv * (p_ref[0, :, 0:64] + p_ref[1, :, 0:64]
                       + xs_ref[:, 0:64]) + b_ref[...])
        out4 = jnp.maximum(pre, 0.0)
        obn4 = _bn(out4, g_ref[...], be_ref[...])
        out4_ref[...] = out4
        obn4_ref[...] = obn4
        logits_ref[...] = jnp.dot(obn4, wc_ref[...],
                                  preferred_element_type=jnp.float32) + bc_ref[...]

    return pl.pallas_call(
        f,
        grid=_GRID,
        in_specs=[_pair_spec(128), _rows_spec(128), _pair_spec(128),
                  _full_spec((1, 64)), _full_spec((1, 64)),
                  _full_spec((1, 64)), _full_spec((64, 10)),
                  _full_spec((1, 10))],
        out_specs=[_rows_spec(10), _rows_spec(64), _rows_spec(64)],
        out_shape=[jax.ShapeDtypeStruct((N, 10), jnp.float32),
                   jax.ShapeDtypeStruct((N, 64), jnp.float32),
                   jax.ShapeDtypeStruct((N, 64), jnp.float32)],
    )(p4, xs4, degp, b4, g4, be4, Wc, bc)


def kernel(x, edge_index, W1, b1, g1, be1, W2, b2, g2, be2,
           W3, b3, g3, be3, W4, b4, g4, be4, Wc, bc):
    src = edge_index[0].astype(jnp.int32)
    dst = edge_index[1].astype(jnp.int32)

    half_shape = (NW, NPH_HALF, NCHP, CH)
    half_slots = NW * NPH_HALF * NCHP * CH
    junk = N + (jnp.arange(half_slots - E, dtype=jnp.int32)
                % (ACC_ROWS - N))
    srcW = _pad_pack(src, half_slots, 0, half_shape)
    dstW = jnp.concatenate([dst, junk]).reshape(half_shape)

    core_slots = NS * NPH2 * NCHP2 * CH2
    core_shape = (NS, NPH2, NCHP2, CH2)
    jspread = N + (jnp.arange(E, dtype=jnp.int32) % (ACC_ROWS - N))
    pad_n = core_slots - E
    pad_src = jnp.arange(pad_n, dtype=jnp.int32) % SLAB
    pad_dst = N + (jnp.arange(pad_n, dtype=jnp.int32) % (ACC_ROWS - N))

    def _core_pack(c):
        off = c * OFF1
        inhalf = (src >= 5000) if c else (src < 5000)
        lidx = jnp.mod(src - off, SLAB)
        dstc = jnp.where(inhalf, dst, jspread)
        sA = jnp.concatenate([lidx, pad_src]).reshape(core_shape)
        dA = jnp.concatenate([dstc, pad_dst]).reshape(core_shape)
        return sA, dA

    s0, d0 = _core_pack(0)
    s1, d1 = _core_pack(1)
    srcS = jnp.stack([s0, s1]).reshape((NW,) + core_shape[1:])
    dstS = jnp.stack([d0, d1]).reshape((NW,) + core_shape[1:])

    ones_tbl = jnp.zeros((8, 128), jnp.float32)
    degp = _deg128(ones_tbl, srcW, dstW)

    b1r, g1r, be1r = b1.reshape(1, -1), g1.reshape(1, -1), be1.reshape(1, -1)
    b2r, g2r, be2r = b2.reshape(1, -1), g2.reshape(1, -1), be2.reshape(1, -1)
    b3r, g3r, be3r = b3.reshape(1, -1), g3.reshape(1, -1), be3.reshape(1, -1)
    b4r, g4r, be4r = b4.reshape(1, -1), g4.reshape(1, -1), be4.reshape(1, -1)
    bcr = bc.reshape(1, -1)

    xs1 = _stage_a(x, degp, W1)
    p1 = _aggS(xs1, srcS, dstS)
    xs2a, xs2b = _stage_b(p1, xs1, degp, b1r, g1r, be1r, W2)
    a2a = _aggS(xs2a, srcS, dstS)
    a2b = _aggS(xs2b, srcS, dstS)
    xs3 = _stage_c(a2a, a2b, xs2a, xs2b, degp, b2r, g2r, be2r, W3)
    p3 = _aggS(xs3, srcS, dstS)
    xs4 = _stage_d(p3, xs3, degp, b3r, g3r, be3r, W4)
    p4 = _aggS(xs4, srcS, dstS)
    logits, out4, obn4 = _stage_e(p4, xs4, degp, b4r, g4r, be4r, Wc, bcr)
    return (logits, out4, obn4)
